# baseline (speedup 1.0000x reference)
import functools

import jax
import jax.numpy as jnp
import numpy as np
from jax import lax
from jax.experimental import pallas as pl
from jax.experimental.pallas import tpu as pltpu
from jax.experimental.pallas import tpu_sc as plsc

N_NODES = 10000
N_EDGES = 320000
D = 128

NC = 2
NS = 16
NW = NC * NS
E_PER_W = N_EDGES // NW
C = 80
NCH = E_PER_W // C
RPT = 640
RPT_LAST = N_NODES - (NS - 1) * RPT

_PERM = np.empty((D,), np.int32)
for _t in range(D // 32):
    for _i in range(16):
        _PERM[32 * _t + 2 * _i] = 32 * _t + _i
        _PERM[32 * _t + 2 * _i + 1] = 32 * _t + 16 + _i


def _gate_body(attr_ref, wp_ref, out_ref):
    z = jnp.dot(attr_ref[...], wp_ref[...],
                preferred_element_type=jnp.float32)
    out_ref[...] = (1.0 / (1.0 + jnp.exp(-z))).astype(jnp.bfloat16)


@jax.jit
def _tc_gate(edge_attr, W_edge_p):
    EB = 4000
    return pl.pallas_call(
        _gate_body,
        grid=(N_EDGES // EB,),
        in_specs=[
            pl.BlockSpec((EB, 4), lambda i: (i, 0)),
            pl.BlockSpec((4, D), lambda i: (0, 0)),
        ],
        out_specs=pl.BlockSpec((EB, D), lambda i: (i, 0)),
        out_shape=jax.ShapeDtypeStruct((N_EDGES, D), jnp.bfloat16),
    )(edge_attr, W_edge_p)


def _sc_body(x_hbm, src_hbm, dst_hbm, gate_hbm, zeros_hbm, out_hbm,
             src_all, dstb, rows_v, gate_a, gate_b, agg_sh,
             gsem0, gsem1, tsem0, tsem1, dsem0, dsem1):
    cid = lax.axis_index("c")
    sid = lax.axis_index("s")
    wid = cid * NS + sid
    ebase = wid * E_PER_W

    pltpu.sync_copy(src_hbm.at[pl.ds(ebase, E_PER_W)], src_all)

    @pl.when(sid < NS - 1)
    def _():
        pltpu.sync_copy(zeros_hbm.at[pl.ds(sid * RPT, RPT)],
                        agg_sh.at[pl.ds(sid * RPT, RPT)])

    @pl.when(sid == NS - 1)
    def _():
        pltpu.sync_copy(zeros_hbm.at[pl.ds((NS - 1) * RPT, RPT_LAST)],
                        agg_sh.at[pl.ds((NS - 1) * RPT, RPT_LAST)])

    plsc.subcore_barrier()

    gsems = (gsem0, gsem1)
    tsems = (tsem0, tsem1)
    dsems = (dsem0, dsem1)
    gates = (gate_a, gate_b)

    def prefetch(i, b):
        pltpu.async_copy(x_hbm.at[src_all.at[pl.ds(i * C, C)]],
                         rows_v.at[b], gsems[b])
        pltpu.async_copy(
            gate_hbm.at[pl.ds((ebase + i * C) * (D // 2), C * (D // 2))],
            gates[b], tsems[b])
        pltpu.async_copy(dst_hbm.at[pl.ds(ebase + i * C, C)],
                         dstb.at[b], dsems[b])

    def compute_scatter(i, b):
        pltpu.make_async_copy(x_hbm.at[src_all.at[pl.ds(0, C)]],
                              rows_v.at[b], gsems[b]).wait()
        pltpu.make_async_copy(gate_hbm.at[pl.ds(0, C * (D // 2))],
                              gates[b], tsems[b]).wait()
        pltpu.make_async_copy(dst_hbm.at[pl.ds(0, C)],
                              dstb.at[b], dsems[b]).wait()
        gv = gates[b]

        @plsc.parallel_loop(0, C, unroll=4)
        def _edges(e):
            eb = e * (D // 2)
            for t in range(D // 32):
                g2 = gv[pl.ds(eb + t * 16, 16)]
                ga = plsc.bitcast(g2 << 16, jnp.float32)
                gb_ = plsc.bitcast(g2 & jnp.int32(-65536), jnp.float32)
                sl0 = pl.ds(t * 32, 16)
                sl1 = pl.ds(t * 32 + 16, 16)
                rows_v[b, e, sl0] = rows_v[b, e, sl0] * ga
                rows_v[b, e, sl1] = rows_v[b, e, sl1] * gb_

        pltpu.sync_copy(rows_v.at[b], agg_sh.at[dstb.at[b]], add=True)

    prefetch(0, 0)

    def pair_body(p, carry):
        i0 = p * 2
        prefetch(i0 + 1, 1)
        compute_scatter(i0, 0)
        prefetch(i0 + 2, 0)
        compute_scatter(i0 + 1, 1)
        return carry

    lax.fori_loop(0, (NCH - 1) // 2, pair_body, 0)
    compute_scatter(NCH - 1, 0)
    plsc.subcore_barrier()

    @pl.when(sid < NS - 1)
    def _():
        pltpu.sync_copy(agg_sh.at[pl.ds(sid * RPT, RPT)],
                        out_hbm.at[pl.ds(cid * N_NODES + sid * RPT, RPT)])

    @pl.when(sid == NS - 1)
    def _():
        pltpu.sync_copy(agg_sh.at[pl.ds((NS - 1) * RPT, RPT_LAST)],
                        out_hbm.at[pl.ds(cid * N_NODES + (NS - 1) * RPT, RPT_LAST)])


@jax.jit
def _sc_aggregate(x, src, dst, gate_flat, zeros):
    mesh = plsc.VectorSubcoreMesh(core_axis_name="c", subcore_axis_name="s")
    return pl.kernel(
        _sc_body,
        mesh=mesh,
        compiler_params=pltpu.CompilerParams(needs_layout_passes=False),
        out_type=jax.ShapeDtypeStruct((NC * N_NODES, D), jnp.float32),
        scratch_types=[
            pltpu.VMEM((E_PER_W,), jnp.int32),
            pltpu.VMEM((2, C), jnp.int32),
            pltpu.VMEM((2, C, D), jnp.float32),
            pltpu.VMEM((C * D // 2,), jnp.int32),
            pltpu.VMEM((C * D // 2,), jnp.int32),
            pltpu.VMEM_SHARED((N_NODES, D), jnp.float32),
            pltpu.SemaphoreType.DMA,
            pltpu.SemaphoreType.DMA,
            pltpu.SemaphoreType.DMA,
            pltpu.SemaphoreType.DMA,
            pltpu.SemaphoreType.DMA,
            pltpu.SemaphoreType.DMA,
        ],
    )(x, src, dst, gate_flat, zeros)


def _tc_body(a0_ref, a1_ref, w_ref, b_ref, out_ref):
    agg = a0_ref[...] + a1_ref[...]
    y = jnp.dot(agg, w_ref[...], preferred_element_type=jnp.float32)
    out_ref[...] = jnp.tanh(y + b_ref[...])


@jax.jit
def _tc_update(agg2, W, b2):
    B = 1000
    nb = N_NODES // B
    return pl.pallas_call(
        _tc_body,
        grid=(nb,),
        in_specs=[
            pl.BlockSpec((B, D), lambda i: (i, 0)),
            pl.BlockSpec((B, D), lambda i: (i + nb, 0)),
            pl.BlockSpec((D, D), lambda i: (0, 0)),
            pl.BlockSpec((1, D), lambda i: (0, 0)),
        ],
        out_specs=pl.BlockSpec((B, D), lambda i: (i, 0)),
        out_shape=jax.ShapeDtypeStruct((N_NODES, D), jnp.float32),
    )(agg2, agg2, W, b2)


def kernel(x, edge_index, edge_attr, W_edge, W, b):
    src = edge_index[0].astype(jnp.int32)
    dst = edge_index[1].astype(jnp.int32)
    gate = _tc_gate(edge_attr, W_edge[:, _PERM])
    gate_i32 = jax.lax.bitcast_convert_type(
        gate.reshape(N_EDGES, D // 2, 2), jnp.int32).reshape(-1)
    zeros = jnp.zeros((N_NODES, D), jnp.float32)
    agg2 = _sc_aggregate(x, src, dst, gate_i32, zeros)
    return _tc_update(agg2, W, b.reshape(1, D))
